# baseline (speedup 1.0000x reference)
import jax
import jax.numpy as jnp
from jax.experimental import pallas as pl
from jax.experimental.pallas import tpu as pltpu

_N, _D = 10000, 128
_C = 2048
_NC = _N // _C
_T = _N - _NC * _C
_STEPS = _NC + 1


def _chunk(i):
    return (i * _C, _C) if i < _NC else (_NC * _C, _T)


def _halves(off, sz):
    h = (sz // 2) // 128 * 128
    return ((off, h), (off + h, sz - h))


def _heads_body(x_hbm, cons_hbm, obj_hbm,
                xv, cv, ov, in_sems, cons_sems, obj_sems):

    def in_copy(i):
        off, sz = _chunk(i)
        return pltpu.make_async_copy(
            x_hbm.at[pl.ds(off, sz), :], xv.at[pl.ds(off, sz), :],
            in_sems.at[i])

    def cons_copies(i):
        off, sz = _chunk(i)
        return [
            pltpu.make_async_copy(
                cv.at[pl.ds(o, s), :], cons_hbm.at[pl.ds(o, s), :],
                cons_sems.at[2 * i + k])
            for k, (o, s) in enumerate(_halves(off, sz))
        ]

    def obj_copies(i):
        off, sz = _chunk(i)
        return [
            pltpu.make_async_copy(
                ov.at[:, pl.ds(o, s)], obj_hbm.at[:, pl.ds(o, s)],
                obj_sems.at[2 * i + k])
            for k, (o, s) in enumerate(_halves(off, sz))
        ]

    for i in range(_STEPS):
        in_copy(i).start()
    for i in range(_STEPS):
        off, sz = _chunk(i)
        in_copy(i).wait()
        xb = xv[pl.ds(off, sz), :]
        e = jnp.exp(xb)
        s = jnp.sum(e, axis=1, keepdims=True)
        cv[pl.ds(off, sz), :] = e / s
        ov[:, pl.ds(off, sz)] = (e / (1.0 + e)).T
        for c in cons_copies(i):
            c.start()
        for c in obj_copies(i):
            c.start()
    for i in range(_STEPS):
        for c in cons_copies(i):
            c.wait()
        for c in obj_copies(i):
            c.wait()


def kernel(x, graph, edge_index):
    del graph, edge_index
    n, d = x.shape
    cons, obj = pl.pallas_call(
        _heads_body,
        in_specs=[pl.BlockSpec(memory_space=pl.ANY)],
        out_specs=[
            pl.BlockSpec(memory_space=pl.ANY),
            pl.BlockSpec(memory_space=pl.ANY),
        ],
        out_shape=[
            jax.ShapeDtypeStruct((n, d), x.dtype),
            jax.ShapeDtypeStruct((d, n), x.dtype),
        ],
        scratch_shapes=[
            pltpu.VMEM((_N, _D), jnp.float32),
            pltpu.VMEM((_N, _D), jnp.float32),
            pltpu.VMEM((_D, _N), jnp.float32),
            pltpu.SemaphoreType.DMA((_STEPS,)),
            pltpu.SemaphoreType.DMA((2 * _STEPS,)),
            pltpu.SemaphoreType.DMA((2 * _STEPS,)),
        ],
    )(x)
    return (cons, obj)


# speedup vs baseline: 1.0122x; 1.0122x over previous
import jax
import jax.numpy as jnp
from jax.experimental import pallas as pl
from jax.experimental.pallas import tpu as pltpu

_N, _D = 10000, 128
_SIZES = (256, 1024, 2048, 2048, 2048, 2048, 528)
_OFFS = tuple(sum(_SIZES[:i]) for i in range(len(_SIZES)))
_STEPS = len(_SIZES)
assert sum(_SIZES) == _N and all(o % 128 == 0 for o in _OFFS)


def _heads_body(x_hbm, cons_hbm, obj_hbm,
                xv, cv, ov, in_sems, cons_sems, obj_sems):

    def in_copy(i):
        off, sz = _OFFS[i], _SIZES[i]
        return pltpu.make_async_copy(
            x_hbm.at[pl.ds(off, sz), :], xv.at[pl.ds(off, sz), :],
            in_sems.at[i])

    def cons_copy(i):
        off, sz = _OFFS[i], _SIZES[i]
        return pltpu.make_async_copy(
            cv.at[pl.ds(off, sz), :], cons_hbm.at[pl.ds(off, sz), :],
            cons_sems.at[i])

    def obj_copy(i):
        off, sz = _OFFS[i], _SIZES[i]
        return pltpu.make_async_copy(
            ov.at[:, pl.ds(off, sz)], obj_hbm.at[:, pl.ds(off, sz)],
            obj_sems.at[i])

    for i in range(_STEPS):
        in_copy(i).start()
    for i in range(_STEPS):
        off, sz = _OFFS[i], _SIZES[i]
        in_copy(i).wait()
        xb = xv[pl.ds(off, sz), :]
        e = jnp.exp(xb)
        s = jnp.sum(e, axis=1, keepdims=True)
        cv[pl.ds(off, sz), :] = e / s
        ov[:, pl.ds(off, sz)] = (e / (1.0 + e)).T
        cons_copy(i).start()
        obj_copy(i).start()
    for i in range(_STEPS):
        cons_copy(i).wait()
        obj_copy(i).wait()


def kernel(x, graph, edge_index):
    del graph, edge_index
    n, d = x.shape
    cons, obj = pl.pallas_call(
        _heads_body,
        in_specs=[pl.BlockSpec(memory_space=pl.ANY)],
        out_specs=[
            pl.BlockSpec(memory_space=pl.ANY),
            pl.BlockSpec(memory_space=pl.ANY),
        ],
        out_shape=[
            jax.ShapeDtypeStruct((n, d), x.dtype),
            jax.ShapeDtypeStruct((d, n), x.dtype),
        ],
        scratch_shapes=[
            pltpu.VMEM((_N, _D), jnp.float32),
            pltpu.VMEM((_N, _D), jnp.float32),
            pltpu.VMEM((_D, _N), jnp.float32),
            pltpu.SemaphoreType.DMA((_STEPS,)),
            pltpu.SemaphoreType.DMA((_STEPS,)),
            pltpu.SemaphoreType.DMA((_STEPS,)),
        ],
    )(x)
    return (cons, obj)
